# baseline (speedup 1.0000x reference)
import functools

import jax
import jax.numpy as jnp
from jax import lax
from jax.experimental import pallas as pl
from jax.experimental.pallas import tpu as pltpu
from jax.experimental.pallas import tpu_sc as plsc

BATCH = 16384
NFACT = 32
NROWS = 1000000
CHUNK = 128
PACK = 128 // NFACT
WIN = 65536
NWIN = (NROWS + WIN - 1) // WIN
PACKED_ROWS = NWIN * (WIN // PACK)


def _relayout_body(u_ref, m_ref, e_ref, uo_ref, mo_ref):
    q = WIN // PACK
    e = e_ref[...]
    for src, dst in ((u_ref, uo_ref), (m_ref, mo_ref)):
        x = src[...]
        acc = jnp.zeros((q, 128), jnp.float32)
        for a in range(PACK):
            acc = acc + jnp.dot(
                x[:, a * q:(a + 1) * q].astype(jnp.bfloat16).T,
                e[:, a * 128:(a + 1) * 128].astype(jnp.bfloat16),
                preferred_element_type=jnp.float32)
        dst[...] = pltpu.bitcast(acc.astype(jnp.bfloat16), jnp.int32)


def _tc_relayout(u_tt, m_tt, e_sel):
    grid = (NWIN,)
    return pl.pallas_call(
        _relayout_body,
        grid=grid,
        compiler_params=pltpu.CompilerParams(
            fuse_transposed_lhs_in_matmul=True,
            vmem_limit_bytes=100 * 1024 * 1024),
        in_specs=[
            pl.BlockSpec((NFACT, WIN), lambda i: (0, i)),
            pl.BlockSpec((NFACT, WIN), lambda i: (0, i)),
            pl.BlockSpec((NFACT, PACK * 128), lambda i: (0, 0)),
        ],
        out_specs=[
            pl.BlockSpec((WIN // PACK // 2, 128), lambda i: (i, 0)),
            pl.BlockSpec((WIN // PACK // 2, 128), lambda i: (i, 0)),
        ],
        out_shape=[
            jax.ShapeDtypeStruct((PACKED_ROWS // 2, 128), jnp.int32),
            jax.ShapeDtypeStruct((PACKED_ROWS // 2, 128), jnp.int32),
        ],
    )(u_tt, m_tt, e_sel)


def _make_sc_gather(num_cores, num_subcores):
    nw = num_cores * num_subcores
    b_per_w = BATCH // nw
    n_chunks = b_per_w // CHUNK
    out_rows_w = b_per_w // PACK
    mesh = plsc.VectorSubcoreMesh(core_axis_name="c", subcore_axis_name="s")

    @functools.partial(
        pl.kernel,
        mesh=mesh,
        compiler_params=pltpu.CompilerParams(needs_layout_passes=False),
        out_type=[
            jax.ShapeDtypeStruct((BATCH // PACK, 128), jnp.float32),
            jax.ShapeDtypeStruct((BATCH // PACK, 128), jnp.float32),
        ],
        scratch_types=[
            pltpu.VMEM((n_chunks, CHUNK), jnp.int32),
            pltpu.VMEM((n_chunks, CHUNK), jnp.int32),
            pltpu.VMEM((n_chunks, CHUNK), jnp.int32),
            pltpu.VMEM((n_chunks, CHUNK), jnp.int32),
            pltpu.VMEM((2, CHUNK, 128), jnp.int32),
            pltpu.VMEM((2, CHUNK, 128), jnp.int32),
            pltpu.VMEM((out_rows_w, 128), jnp.float32),
            pltpu.VMEM((out_rows_w, 128), jnp.float32),
            pltpu.SemaphoreType.DMA,
            pltpu.SemaphoreType.DMA,
            pltpu.SemaphoreType.DMA,
            pltpu.SemaphoreType.DMA,
        ],
    )
    def sc_gather(users_hbm, movies_hbm, ut_hbm, mt_hbm, uo_hbm, mo_hbm,
                  uraw, mraw, uprow, mprow, ubuf, mbuf, uout, mout,
                  su0, su1, sm0, sm1):
        sems_u = (su0, su1)
        sems_m = (sm0, sm1)
        wid = lax.axis_index("s") * num_cores + lax.axis_index("c")
        pltpu.sync_copy(users_hbm.at[wid], uraw)
        pltpu.sync_copy(movies_hbm.at[wid], mraw)
        for j in range(n_chunks):
            for t in range(CHUNK // 16):
                s = pl.ds(t * 16, 16)
                ru = uraw[j, s]
                rm = mraw[j, s]
                pu = ((lax.shift_right_logical(ru, 16) << 14) + (ru & 16383))
                pm = ((lax.shift_right_logical(rm, 16) << 14) + (rm & 16383))
                uprow[j, s] = lax.shift_right_logical(pu, 1)
                mprow[j, s] = lax.shift_right_logical(pm, 1)

        def start(j):
            slot = j % 2
            cu = pltpu.async_copy(ut_hbm.at[uprow.at[j]], ubuf.at[slot],
                                  sems_u[slot])
            cm = pltpu.async_copy(mt_hbm.at[mprow.at[j]], mbuf.at[slot],
                                  sems_m[slot])
            return cu, cm

        def extract(j, raw, buf, out):
            slot = j % 2

            def tbody(t, carry):
                iv = raw[j, pl.ds(t * 16, 16)]
                ov = (lax.shift_right_logical(iv, 14) & 3) << 5
                sv = iv & 1
                for l in range(16):
                    o = ov[l]
                    sel = sv[l]
                    r = t * 16 + l
                    orow = j * (CHUNK // PACK) + t * 4 + (l >> 2)
                    ocol = (l & 3) * NFACT
                    for h in range(2):
                        w = buf[slot, r, pl.ds(o + h * 16, 16)]
                        lo = w << 16
                        hi = w & jnp.int32(-65536)
                        bits = jnp.where(sel == 0, lo, hi)
                        out[orow, pl.ds(ocol + h * 16, 16)] = plsc.bitcast(
                            bits, jnp.float32)
                return carry

            lax.fori_loop(0, CHUNK // 16, tbody, 0)

        pend = start(0)
        for j in range(n_chunks):
            cu, cm = pend
            if j + 1 < n_chunks:
                pend = start(j + 1)
            cu.wait()
            extract(j, uraw, ubuf, uout)
            cm.wait()
            extract(j, mraw, mbuf, mout)

        base = wid * out_rows_w
        pltpu.sync_copy(uout, uo_hbm.at[pl.ds(base, out_rows_w)])
        pltpu.sync_copy(mout, mo_hbm.at[pl.ds(base, out_rows_w)])

    return sc_gather


def _mlp_body(u_ref, m_ref, a_ref, b_ref, c_ref, b1_ref, w2_ref, b2_ref, o_ref):
    u = u_ref[...]
    m = m_ref[...]
    e = u * m
    h = (jnp.dot(e, a_ref[...], preferred_element_type=jnp.float32)
         + jnp.dot(u, b_ref[...], preferred_element_type=jnp.float32)
         + jnp.dot(m, c_ref[...], preferred_element_type=jnp.float32)
         + b1_ref[...])
    h = jnp.maximum(h, 0.0)
    o = jnp.dot(h, w2_ref[...], preferred_element_type=jnp.float32) + b2_ref[...]
    o_ref[...] = jax.nn.sigmoid(o)


def _tc_mlp(u128, m128, a_bd, b_bd, c_bd, b1t, w2_bd, b2t):
    rows = 512
    grid = ((BATCH // PACK) // rows,)
    wspec = lambda shape: pl.BlockSpec(shape, lambda i: (0, 0))
    return pl.pallas_call(
        _mlp_body,
        grid=grid,
        in_specs=[
            pl.BlockSpec((rows, 128), lambda i: (i, 0)),
            pl.BlockSpec((rows, 128), lambda i: (i, 0)),
            wspec((128, PACK * 8)),
            wspec((128, PACK * 8)),
            wspec((128, PACK * 8)),
            wspec((1, PACK * 8)),
            wspec((PACK * 8, PACK)),
            wspec((1, PACK)),
        ],
        out_specs=pl.BlockSpec((rows, PACK), lambda i: (i, 0)),
        out_shape=jax.ShapeDtypeStruct((BATCH // PACK, PACK), jnp.float32),
    )(u128, m128, a_bd, b_bd, c_bd, b1t, w2_bd, b2t)


def kernel(users, movies, user_table, movie_table, W1, b1, W2, b2):
    info = plsc.get_sparse_core_info()
    nc, ns = info.num_cores, info.num_subcores
    nw = nc * ns
    b_per_w = BATCH // nw
    n_chunks = b_per_w // CHUNK
    i32eye = jnp.eye(NFACT, dtype=jnp.float32)
    e_sel = jnp.zeros((NFACT, PACK * 128), jnp.float32)
    for a in range(PACK):
        s = a * 128 + a * NFACT
        e_sel = e_sel.at[:, s:s + NFACT].set(i32eye)
    ut_c, mt_c = _tc_relayout(user_table.T, movie_table.T, e_sel)
    sc_gather = _make_sc_gather(nc, ns)
    users_r = users.astype(jnp.int32).reshape(nw, n_chunks, CHUNK)
    movies_r = movies.astype(jnp.int32).reshape(nw, n_chunks, CHUNK)
    u128, m128 = sc_gather(users_r, movies_r, ut_c, mt_c)

    eye = jnp.eye(PACK, dtype=jnp.float32)
    a_bd = jnp.kron(eye, W1[0:NFACT])
    b_bd = jnp.kron(eye, W1[NFACT:2 * NFACT])
    c_bd = jnp.kron(eye, W1[2 * NFACT:3 * NFACT])
    w2_bd = jnp.kron(eye, W2)
    b1t = jnp.tile(b1, PACK).reshape(1, PACK * 8)
    b2t = jnp.broadcast_to(b2.reshape(1, 1), (1, PACK))
    out = _tc_mlp(u128, m128, a_bd, b_bd, c_bd, b1t, w2_bd, b2t)
    return out.reshape(BATCH, 1)


# speedup vs baseline: 1.0136x; 1.0136x over previous
import functools

import jax
import jax.numpy as jnp
from jax import lax
from jax.experimental import pallas as pl
from jax.experimental.pallas import tpu as pltpu
from jax.experimental.pallas import tpu_sc as plsc

BATCH = 16384
NFACT = 32
NROWS = 1000000
CHUNK = 128
PACK = 128 // NFACT
WIN = 32768
NWIN = (NROWS + WIN - 1) // WIN
PACKED_ROWS = NWIN * (WIN // PACK)


def _relayout_body(u_ref, m_ref, e_ref, uo_ref, mo_ref):
    q = WIN // PACK
    e = e_ref[...]
    for src, dst in ((u_ref, uo_ref), (m_ref, mo_ref)):
        x = src[...]
        acc = jnp.zeros((q, 128), jnp.float32)
        for a in range(PACK):
            acc = acc + jnp.dot(
                x[:, a * q:(a + 1) * q].astype(jnp.bfloat16).T,
                e[:, a * 128:(a + 1) * 128].astype(jnp.bfloat16),
                preferred_element_type=jnp.float32)
        dst[...] = pltpu.bitcast(acc.astype(jnp.bfloat16), jnp.int32)


def _tc_relayout(u_tt, m_tt, e_sel):
    grid = (NWIN,)
    return pl.pallas_call(
        _relayout_body,
        grid=grid,
        compiler_params=pltpu.CompilerParams(
            fuse_transposed_lhs_in_matmul=True),
        in_specs=[
            pl.BlockSpec((NFACT, WIN), lambda i: (0, i)),
            pl.BlockSpec((NFACT, WIN), lambda i: (0, i)),
            pl.BlockSpec((NFACT, PACK * 128), lambda i: (0, 0)),
        ],
        out_specs=[
            pl.BlockSpec((WIN // PACK // 2, 128), lambda i: (i, 0)),
            pl.BlockSpec((WIN // PACK // 2, 128), lambda i: (i, 0)),
        ],
        out_shape=[
            jax.ShapeDtypeStruct((PACKED_ROWS // 2, 128), jnp.int32),
            jax.ShapeDtypeStruct((PACKED_ROWS // 2, 128), jnp.int32),
        ],
    )(u_tt, m_tt, e_sel)


def _make_sc_gather(num_cores, num_subcores):
    nw = num_cores * num_subcores
    b_per_w = BATCH // nw
    n_chunks = b_per_w // CHUNK
    out_rows_w = b_per_w // PACK
    mesh = plsc.VectorSubcoreMesh(core_axis_name="c", subcore_axis_name="s")

    @functools.partial(
        pl.kernel,
        mesh=mesh,
        compiler_params=pltpu.CompilerParams(needs_layout_passes=False),
        out_type=[
            jax.ShapeDtypeStruct((BATCH // PACK, 128), jnp.float32),
            jax.ShapeDtypeStruct((BATCH // PACK, 128), jnp.float32),
        ],
        scratch_types=[
            pltpu.VMEM((n_chunks, CHUNK), jnp.int32),
            pltpu.VMEM((n_chunks, CHUNK), jnp.int32),
            pltpu.VMEM((n_chunks, CHUNK), jnp.int32),
            pltpu.VMEM((n_chunks, CHUNK), jnp.int32),
            pltpu.VMEM((2, CHUNK, 128), jnp.int32),
            pltpu.VMEM((2, CHUNK, 128), jnp.int32),
            pltpu.VMEM((out_rows_w, 128), jnp.float32),
            pltpu.VMEM((out_rows_w, 128), jnp.float32),
            pltpu.SemaphoreType.DMA,
            pltpu.SemaphoreType.DMA,
            pltpu.SemaphoreType.DMA,
            pltpu.SemaphoreType.DMA,
        ],
    )
    def sc_gather(users_hbm, movies_hbm, ut_hbm, mt_hbm, uo_hbm, mo_hbm,
                  uraw, mraw, uprow, mprow, ubuf, mbuf, uout, mout,
                  su0, su1, sm0, sm1):
        sems_u = (su0, su1)
        sems_m = (sm0, sm1)
        wid = lax.axis_index("s") * num_cores + lax.axis_index("c")
        pltpu.sync_copy(users_hbm.at[wid], uraw)
        pltpu.sync_copy(movies_hbm.at[wid], mraw)
        for j in range(n_chunks):
            for t in range(CHUNK // 16):
                s = pl.ds(t * 16, 16)
                ru = uraw[j, s]
                rm = mraw[j, s]
                pu = ((lax.shift_right_logical(ru, 15) << 13) + (ru & 8191))
                pm = ((lax.shift_right_logical(rm, 15) << 13) + (rm & 8191))
                uprow[j, s] = lax.shift_right_logical(pu, 1)
                mprow[j, s] = lax.shift_right_logical(pm, 1)

        def start(j):
            slot = j % 2
            cu = pltpu.async_copy(ut_hbm.at[uprow.at[j]], ubuf.at[slot],
                                  sems_u[slot])
            cm = pltpu.async_copy(mt_hbm.at[mprow.at[j]], mbuf.at[slot],
                                  sems_m[slot])
            return cu, cm

        def extract(j, raw, buf, out):
            slot = j % 2

            def tbody(t, carry):
                iv = raw[j, pl.ds(t * 16, 16)]
                ov = (lax.shift_right_logical(iv, 13) & 3) << 5
                sv = iv & 1
                for l in range(16):
                    o = ov[l]
                    sel = sv[l]
                    r = t * 16 + l
                    orow = j * (CHUNK // PACK) + t * 4 + (l >> 2)
                    ocol = (l & 3) * NFACT
                    for h in range(2):
                        w = buf[slot, r, pl.ds(o + h * 16, 16)]
                        lo = w << 16
                        hi = w & jnp.int32(-65536)
                        bits = jnp.where(sel == 0, lo, hi)
                        out[orow, pl.ds(ocol + h * 16, 16)] = plsc.bitcast(
                            bits, jnp.float32)
                return carry

            lax.fori_loop(0, CHUNK // 16, tbody, 0)

        pend = start(0)
        for j in range(n_chunks):
            cu, cm = pend
            if j + 1 < n_chunks:
                pend = start(j + 1)
            cu.wait()
            extract(j, uraw, ubuf, uout)
            cm.wait()
            extract(j, mraw, mbuf, mout)

        base = wid * out_rows_w
        pltpu.sync_copy(uout, uo_hbm.at[pl.ds(base, out_rows_w)])
        pltpu.sync_copy(mout, mo_hbm.at[pl.ds(base, out_rows_w)])

    return sc_gather


def _mlp_body(u_ref, m_ref, a_ref, b_ref, c_ref, b1_ref, w2_ref, b2_ref, o_ref):
    u = u_ref[...]
    m = m_ref[...]
    e = u * m
    h = (jnp.dot(e, a_ref[...], preferred_element_type=jnp.float32)
         + jnp.dot(u, b_ref[...], preferred_element_type=jnp.float32)
         + jnp.dot(m, c_ref[...], preferred_element_type=jnp.float32)
         + b1_ref[...])
    h = jnp.maximum(h, 0.0)
    o = jnp.dot(h, w2_ref[...], preferred_element_type=jnp.float32) + b2_ref[...]
    o_ref[...] = jax.nn.sigmoid(o)


def _tc_mlp(u128, m128, a_bd, b_bd, c_bd, b1t, w2_bd, b2t):
    rows = 512
    grid = ((BATCH // PACK) // rows,)
    wspec = lambda shape: pl.BlockSpec(shape, lambda i: (0, 0))
    return pl.pallas_call(
        _mlp_body,
        grid=grid,
        in_specs=[
            pl.BlockSpec((rows, 128), lambda i: (i, 0)),
            pl.BlockSpec((rows, 128), lambda i: (i, 0)),
            wspec((128, PACK * 8)),
            wspec((128, PACK * 8)),
            wspec((128, PACK * 8)),
            wspec((1, PACK * 8)),
            wspec((PACK * 8, PACK)),
            wspec((1, PACK)),
        ],
        out_specs=pl.BlockSpec((rows, PACK), lambda i: (i, 0)),
        out_shape=jax.ShapeDtypeStruct((BATCH // PACK, PACK), jnp.float32),
    )(u128, m128, a_bd, b_bd, c_bd, b1t, w2_bd, b2t)


def kernel(users, movies, user_table, movie_table, W1, b1, W2, b2):
    info = plsc.get_sparse_core_info()
    nc, ns = info.num_cores, info.num_subcores
    nw = nc * ns
    b_per_w = BATCH // nw
    n_chunks = b_per_w // CHUNK
    i32eye = jnp.eye(NFACT, dtype=jnp.float32)
    e_sel = jnp.zeros((NFACT, PACK * 128), jnp.float32)
    for a in range(PACK):
        s = a * 128 + a * NFACT
        e_sel = e_sel.at[:, s:s + NFACT].set(i32eye)
    ut_c, mt_c = _tc_relayout(user_table.T, movie_table.T, e_sel)
    sc_gather = _make_sc_gather(nc, ns)
    users_r = users.astype(jnp.int32).reshape(nw, n_chunks, CHUNK)
    movies_r = movies.astype(jnp.int32).reshape(nw, n_chunks, CHUNK)
    u128, m128 = sc_gather(users_r, movies_r, ut_c, mt_c)

    eye = jnp.eye(PACK, dtype=jnp.float32)
    a_bd = jnp.kron(eye, W1[0:NFACT])
    b_bd = jnp.kron(eye, W1[NFACT:2 * NFACT])
    c_bd = jnp.kron(eye, W1[2 * NFACT:3 * NFACT])
    w2_bd = jnp.kron(eye, W2)
    b1t = jnp.tile(b1, PACK).reshape(1, PACK * 8)
    b2t = jnp.broadcast_to(b2.reshape(1, 1), (1, PACK))
    out = _tc_mlp(u128, m128, a_bd, b_bd, c_bd, b1t, w2_bd, b2t)
    return out.reshape(BATCH, 1)
